# baseline (speedup 1.0000x reference)
import jax
import jax.numpy as jnp
from jax.experimental import pallas as pl
from jax.experimental.pallas import tpu as pltpu

_B, _L, _D, _E, _F, _TOPK = 1, 2048, 1024, 8, 4096, 2
_N = _B * _L
_FT = 1024
_NF = _F // _FT
_BLK = 640
_NGB = 14
_C = 256


def _router_meta_body(x_ref, gw_ref, logits_ref, pos0_ref, pos1_ref,
                      wv0_ref, wv1_ref, be_ref, rs_ref):
    x = x_ref[...]
    gw = gw_ref[...]
    logits = jax.lax.dot_general(
        x, gw, (((1,), (0,)), ((), ())),
        precision=jax.lax.Precision.DEFAULT,
        preferred_element_type=jnp.float32)
    logits_ref[...] = logits

    eidx = jax.lax.broadcasted_iota(jnp.int32, (_N, _E), 1)
    m1 = jnp.max(logits, axis=1, keepdims=True)
    i1 = jnp.min(jnp.where(logits == m1, eidx, _E), axis=1, keepdims=True)
    masked = jnp.where(eidx == i1, -jnp.inf, logits)
    m2 = jnp.max(masked, axis=1, keepdims=True)
    i2 = jnp.min(jnp.where(masked == m2, eidx, _E), axis=1, keepdims=True)
    t = jnp.exp(m2 - m1)
    denom = 1.0 + t
    wv0_ref[...] = 1.0 / denom
    wv1_ref[...] = t / denom

    one = jnp.float32(1.0)
    zero = jnp.float32(0.0)
    ohAf = jnp.where(eidx == i1, one, zero)
    ohBf = jnp.where(eidx == i2, one, zero)
    oh = ohAf + ohBf

    ri = jax.lax.broadcasted_iota(jnp.int32, (_C, _C), 0)
    ci = jax.lax.broadcasted_iota(jnp.int32, (_C, _C), 1)
    t256 = jnp.where(ci < ri, one, zero).astype(jnp.bfloat16)
    base = jnp.zeros((1, _E), jnp.float32)
    rows = []
    for c in range(_N // _C):
        ohc = oh[c * _C:(c + 1) * _C]
        local = jax.lax.dot_general(
            t256, ohc.astype(jnp.bfloat16), (((1,), (0,)), ((), ())),
            preferred_element_type=jnp.float32)
        rows.append(local + base)
        base = base + jnp.sum(ohc, axis=0, keepdims=True)
    csum = jnp.concatenate(rows, axis=0)

    rank0 = jnp.sum(csum * ohAf, axis=1, keepdims=True)
    rank1 = jnp.sum(csum * ohBf, axis=1, keepdims=True)

    counts = base.astype(jnp.int32)
    nb = (counts + (_BLK - 1)) // _BLK
    inc = nb
    for sh in (1, 2, 4):
        shifted = jnp.concatenate(
            [jnp.zeros((1, sh), jnp.int32), inc[:, :-sh]], axis=1)
        inc = inc + shifted
    cnb = inc - nb
    off_rows = cnb * _BLK

    zi = jnp.int32(0)
    off0 = jnp.sum(jnp.where(eidx == i1, off_rows, zi), axis=1, keepdims=True)
    off1 = jnp.sum(jnp.where(eidx == i2, off_rows, zi), axis=1, keepdims=True)
    pos0_ref[...] = off0 + rank0.astype(jnp.int32)
    pos1_ref[...] = off1 + rank1.astype(jnp.int32)

    gbi = jax.lax.broadcasted_iota(jnp.int32, (1, _NGB), 1)
    eg = jnp.zeros((1, _NGB), jnp.int32)
    rs = jnp.zeros((1, _NGB), jnp.int32)
    totnb = inc[0, _E - 1]
    elast = jnp.int32(0)
    for e in range(_E):
        cnb_e = cnb[0, e]
        nb_e = nb[0, e]
        off_e = off_rows[0, e]
        sel = (gbi >= cnb_e) & (gbi < cnb_e + nb_e)
        eg = jnp.where(sel, e, eg)
        rs = jnp.where(sel, off_e + (gbi - cnb_e) * _BLK, rs)
        elast = jnp.where(nb_e > 0, jnp.int32(e), elast)
    valid = gbi < totnb
    be_ref[...] = jnp.where(valid, eg, elast)
    rs_ref[...] = jnp.where(valid, rs, -1)


def _ffn_body(be_ref, rs_ref, xb_ref, p0r_ref, p1r_ref, w0r_ref, w1r_ref,
              w1_ref, w2_ref, out_ref, xb_s, y_s, dw_s):
    gb = pl.program_id(0)
    f = pl.program_id(1)
    rbase = rs_ref[gb]

    @pl.when(rbase >= 0)
    def _():
        f1 = jnp.float32(1.0)
        f0 = jnp.float32(0.0)

        @pl.when(f == 0)
        def _():
            riota = jax.lax.broadcasted_iota(jnp.int32, (_BLK, _N), 0) + rbase
            m0 = p0r_ref[...] == riota
            m1 = p1r_ref[...] == riota
            disp = jnp.where(m0 | m1, f1, f0).astype(jnp.bfloat16)
            dw_s[...] = (jnp.where(m0, w0r_ref[...], f0)
                         + jnp.where(m1, w1r_ref[...], f0)
                         ).astype(jnp.bfloat16)
            xg = jax.lax.dot_general(
                disp, xb_ref[...], (((1,), (0,)), ((), ())),
                preferred_element_type=jnp.float32)
            xb_s[...] = xg.astype(jnp.bfloat16)

        xbb = xb_s[...]
        w1t = w1_ref[0].astype(jnp.bfloat16)
        h = jax.lax.dot_general(
            xbb, w1t, (((1,), (0,)), ((), ())),
            preferred_element_type=jnp.float32)
        h = h * jax.lax.logistic(h)
        h = h.astype(jnp.bfloat16)
        w2t = w2_ref[0].astype(jnp.bfloat16)
        contrib = jax.lax.dot_general(
            h, w2t, (((1,), (0,)), ((), ())),
            preferred_element_type=jnp.float32)

        @pl.when(f == 0)
        def _():
            y_s[...] = contrib

        @pl.when(f > 0)
        def _():
            y_s[...] = y_s[...] + contrib

        @pl.when(f == _NF - 1)
        def _():
            y_sc = y_s[...].astype(jnp.bfloat16)
            res = jax.lax.dot_general(
                dw_s[...], y_sc, (((0,), (0,)), ((), ())),
                preferred_element_type=jnp.float32)

            @pl.when(gb == 0)
            def _():
                out_ref[...] = res

            @pl.when(gb > 0)
            def _():
                out_ref[...] += res


def kernel(x, gate_w, w1, w2):
    x_flat = x.reshape(_N, _D)
    logits, pos0, pos1, wv0, wv1, be, rs = pl.pallas_call(
        _router_meta_body,
        out_shape=(
            jax.ShapeDtypeStruct((_N, _E), jnp.float32),
            jax.ShapeDtypeStruct((_N, 1), jnp.int32),
            jax.ShapeDtypeStruct((_N, 1), jnp.int32),
            jax.ShapeDtypeStruct((_N, 1), jnp.float32),
            jax.ShapeDtypeStruct((_N, 1), jnp.float32),
            jax.ShapeDtypeStruct((1, _NGB), jnp.int32),
            jax.ShapeDtypeStruct((1, _NGB), jnp.int32),
        ),
    )(x_flat, gate_w)

    xb = x_flat.astype(jnp.bfloat16)
    p0r = pos0.reshape(1, _N)
    p1r = pos1.reshape(1, _N)
    w0r = wv0.reshape(1, _N)
    w1r = wv1.reshape(1, _N)

    full = lambda gb, f, be_, rs_: (0, 0)

    def _w1_map(gb, f, be_, rs_):
        fx = jnp.where(rs_[gb] >= 0, f, _NF - 1)
        return (be_[gb], 0, fx)

    def _w2_map(gb, f, be_, rs_):
        fx = jnp.where(rs_[gb] >= 0, f, _NF - 1)
        return (be_[gb], fx, 0)

    grid_spec = pltpu.PrefetchScalarGridSpec(
        num_scalar_prefetch=2,
        grid=(_NGB, _NF),
        in_specs=[
            pl.BlockSpec((_N, _D), full),
            pl.BlockSpec((1, _N), full),
            pl.BlockSpec((1, _N), full),
            pl.BlockSpec((1, _N), full),
            pl.BlockSpec((1, _N), full),
            pl.BlockSpec((1, _D, _FT), _w1_map),
            pl.BlockSpec((1, _FT, _D), _w2_map),
        ],
        out_specs=pl.BlockSpec((_N, _D), full),
        scratch_shapes=[
            pltpu.VMEM((_BLK, _D), jnp.bfloat16),
            pltpu.VMEM((_BLK, _D), jnp.float32),
            pltpu.VMEM((_BLK, _N), jnp.bfloat16),
        ],
    )
    out = pl.pallas_call(
        _ffn_body,
        grid_spec=grid_spec,
        out_shape=jax.ShapeDtypeStruct((_N, _D), jnp.float32),
        compiler_params=pltpu.CompilerParams(
            dimension_semantics=("arbitrary", "arbitrary")),
    )(be.reshape(_NGB), rs.reshape(_NGB),
      xb, p0r, p1r, w0r, w1r, w1, w2)
    return out.reshape(_B, _L, _D), logits


# speedup vs baseline: 1.0475x; 1.0475x over previous
import jax
import jax.numpy as jnp
from jax.experimental import pallas as pl
from jax.experimental.pallas import tpu as pltpu

_B, _L, _D, _E, _F, _TOPK = 1, 2048, 1024, 8, 4096, 2
_N = _B * _L
_FT = 2048
_NF = _F // _FT
_BLK = 640
_NGB = 14
_C = 256


def _router_meta_body(x_ref, gw_ref, logits_ref, pos0_ref, pos1_ref,
                      wv0_ref, wv1_ref, be_ref, rs_ref):
    x = x_ref[...]
    gw = gw_ref[...]
    logits = jax.lax.dot_general(
        x, gw, (((1,), (0,)), ((), ())),
        precision=jax.lax.Precision.DEFAULT,
        preferred_element_type=jnp.float32)
    logits_ref[...] = logits

    eidx = jax.lax.broadcasted_iota(jnp.int32, (_N, _E), 1)
    m1 = jnp.max(logits, axis=1, keepdims=True)
    i1 = jnp.min(jnp.where(logits == m1, eidx, _E), axis=1, keepdims=True)
    masked = jnp.where(eidx == i1, -jnp.inf, logits)
    m2 = jnp.max(masked, axis=1, keepdims=True)
    i2 = jnp.min(jnp.where(masked == m2, eidx, _E), axis=1, keepdims=True)
    t = jnp.exp(m2 - m1)
    denom = 1.0 + t
    wv0_ref[...] = 1.0 / denom
    wv1_ref[...] = t / denom

    one = jnp.float32(1.0)
    zero = jnp.float32(0.0)
    ohAf = jnp.where(eidx == i1, one, zero)
    ohBf = jnp.where(eidx == i2, one, zero)
    oh = ohAf + ohBf

    ri = jax.lax.broadcasted_iota(jnp.int32, (_C, _C), 0)
    ci = jax.lax.broadcasted_iota(jnp.int32, (_C, _C), 1)
    t256 = jnp.where(ci < ri, one, zero).astype(jnp.bfloat16)
    base = jnp.zeros((1, _E), jnp.float32)
    rows = []
    for c in range(_N // _C):
        ohc = oh[c * _C:(c + 1) * _C]
        local = jax.lax.dot_general(
            t256, ohc.astype(jnp.bfloat16), (((1,), (0,)), ((), ())),
            preferred_element_type=jnp.float32)
        rows.append(local + base)
        base = base + jnp.sum(ohc, axis=0, keepdims=True)
    csum = jnp.concatenate(rows, axis=0)

    rank0 = jnp.sum(csum * ohAf, axis=1, keepdims=True)
    rank1 = jnp.sum(csum * ohBf, axis=1, keepdims=True)

    counts = base.astype(jnp.int32)
    nb = (counts + (_BLK - 1)) // _BLK
    inc = nb
    for sh in (1, 2, 4):
        shifted = jnp.concatenate(
            [jnp.zeros((1, sh), jnp.int32), inc[:, :-sh]], axis=1)
        inc = inc + shifted
    cnb = inc - nb
    off_rows = cnb * _BLK

    zi = jnp.int32(0)
    off0 = jnp.sum(jnp.where(eidx == i1, off_rows, zi), axis=1, keepdims=True)
    off1 = jnp.sum(jnp.where(eidx == i2, off_rows, zi), axis=1, keepdims=True)
    pos0_ref[...] = off0 + rank0.astype(jnp.int32)
    pos1_ref[...] = off1 + rank1.astype(jnp.int32)

    gbi = jax.lax.broadcasted_iota(jnp.int32, (1, _NGB), 1)
    eg = jnp.zeros((1, _NGB), jnp.int32)
    rs = jnp.zeros((1, _NGB), jnp.int32)
    totnb = inc[0, _E - 1]
    elast = jnp.int32(0)
    for e in range(_E):
        cnb_e = cnb[0, e]
        nb_e = nb[0, e]
        off_e = off_rows[0, e]
        sel = (gbi >= cnb_e) & (gbi < cnb_e + nb_e)
        eg = jnp.where(sel, e, eg)
        rs = jnp.where(sel, off_e + (gbi - cnb_e) * _BLK, rs)
        elast = jnp.where(nb_e > 0, jnp.int32(e), elast)
    valid = gbi < totnb
    be_ref[...] = jnp.where(valid, eg, elast)
    rs_ref[...] = jnp.where(valid, rs, -1)


def _ffn_body(be_ref, rs_ref, xb_ref, p0r_ref, p1r_ref, w0r_ref, w1r_ref,
              w1_ref, w2_ref, out_ref, xb_s, y_s, dw_s):
    gb = pl.program_id(0)
    f = pl.program_id(1)
    rbase = rs_ref[gb]

    @pl.when(rbase >= 0)
    def _():
        f1 = jnp.float32(1.0)
        f0 = jnp.float32(0.0)

        @pl.when(f == 0)
        def _():
            riota = jax.lax.broadcasted_iota(jnp.int32, (_BLK, _N), 0) + rbase
            m0 = p0r_ref[...] == riota
            m1 = p1r_ref[...] == riota
            disp = jnp.where(m0 | m1, f1, f0).astype(jnp.bfloat16)
            dw_s[...] = (jnp.where(m0, w0r_ref[...], f0)
                         + jnp.where(m1, w1r_ref[...], f0)
                         ).astype(jnp.bfloat16)
            xg = jax.lax.dot_general(
                disp, xb_ref[...], (((1,), (0,)), ((), ())),
                preferred_element_type=jnp.float32)
            xb_s[...] = xg.astype(jnp.bfloat16)

        xbb = xb_s[...]
        w1t = w1_ref[0].astype(jnp.bfloat16)
        h = jax.lax.dot_general(
            xbb, w1t, (((1,), (0,)), ((), ())),
            preferred_element_type=jnp.float32)
        h = h * jax.lax.logistic(h)
        h = h.astype(jnp.bfloat16)
        w2t = w2_ref[0].astype(jnp.bfloat16)
        contrib = jax.lax.dot_general(
            h, w2t, (((1,), (0,)), ((), ())),
            preferred_element_type=jnp.float32)

        @pl.when(f == 0)
        def _():
            y_s[...] = contrib

        @pl.when(f > 0)
        def _():
            y_s[...] = y_s[...] + contrib

        @pl.when(f == _NF - 1)
        def _():
            y_sc = y_s[...].astype(jnp.bfloat16)
            res = jax.lax.dot_general(
                dw_s[...], y_sc, (((0,), (0,)), ((), ())),
                preferred_element_type=jnp.float32)

            @pl.when(gb == 0)
            def _():
                out_ref[...] = res

            @pl.when(gb > 0)
            def _():
                out_ref[...] += res


def kernel(x, gate_w, w1, w2):
    x_flat = x.reshape(_N, _D)
    logits, pos0, pos1, wv0, wv1, be, rs = pl.pallas_call(
        _router_meta_body,
        out_shape=(
            jax.ShapeDtypeStruct((_N, _E), jnp.float32),
            jax.ShapeDtypeStruct((_N, 1), jnp.int32),
            jax.ShapeDtypeStruct((_N, 1), jnp.int32),
            jax.ShapeDtypeStruct((_N, 1), jnp.float32),
            jax.ShapeDtypeStruct((_N, 1), jnp.float32),
            jax.ShapeDtypeStruct((1, _NGB), jnp.int32),
            jax.ShapeDtypeStruct((1, _NGB), jnp.int32),
        ),
    )(x_flat, gate_w)

    xb = x_flat.astype(jnp.bfloat16)
    p0r = pos0.reshape(1, _N)
    p1r = pos1.reshape(1, _N)
    w0r = wv0.reshape(1, _N)
    w1r = wv1.reshape(1, _N)

    full = lambda gb, f, be_, rs_: (0, 0)

    def _w1_map(gb, f, be_, rs_):
        fx = jnp.where(rs_[gb] >= 0, f, _NF - 1)
        return (be_[gb], 0, fx)

    def _w2_map(gb, f, be_, rs_):
        fx = jnp.where(rs_[gb] >= 0, f, _NF - 1)
        return (be_[gb], fx, 0)

    grid_spec = pltpu.PrefetchScalarGridSpec(
        num_scalar_prefetch=2,
        grid=(_NGB, _NF),
        in_specs=[
            pl.BlockSpec((_N, _D), full),
            pl.BlockSpec((1, _N), full),
            pl.BlockSpec((1, _N), full),
            pl.BlockSpec((1, _N), full),
            pl.BlockSpec((1, _N), full),
            pl.BlockSpec((1, _D, _FT), _w1_map),
            pl.BlockSpec((1, _FT, _D), _w2_map),
        ],
        out_specs=pl.BlockSpec((_N, _D), full),
        scratch_shapes=[
            pltpu.VMEM((_BLK, _D), jnp.bfloat16),
            pltpu.VMEM((_BLK, _D), jnp.float32),
            pltpu.VMEM((_BLK, _N), jnp.bfloat16),
        ],
    )
    out = pl.pallas_call(
        _ffn_body,
        grid_spec=grid_spec,
        out_shape=jax.ShapeDtypeStruct((_N, _D), jnp.float32),
        compiler_params=pltpu.CompilerParams(
            dimension_semantics=("arbitrary", "arbitrary"),
            vmem_limit_bytes=100 * 1024 * 1024),
    )(be.reshape(_NGB), rs.reshape(_NGB),
      xb, p0r, p1r, w0r, w1r, w1, w2)
    return out.reshape(_B, _L, _D), logits
